# baseline (speedup 1.0000x reference)
import functools

import jax
import jax.numpy as jnp
from jax import lax
from jax.experimental import pallas as pl
from jax.experimental.pallas import tpu as pltpu
from jax.experimental.pallas import tpu_sc as plsc

BATCH, SEQ, D = 4096, 200, 64
N = BATCH * SEQ
NC, NS = 2, 16
NW = NC * NS
PER_W = N // NW
CHUNK = 128
STEPS = PER_W // CHUNK

_MESH = plsc.VectorSubcoreMesh(core_axis_name="c", subcore_axis_name="s")


@functools.partial(
    pl.kernel,
    out_type=jax.ShapeDtypeStruct((N, D), jnp.float32),
    mesh=_MESH,
    scratch_types=[
        pltpu.VMEM((CHUNK,), jnp.int32),
        pltpu.VMEM((CHUNK, D), jnp.float32),
        pltpu.SemaphoreType.DMA,
    ],
    compiler_params=pltpu.CompilerParams(use_tc_tiling_on_sc=False),
)
def _sc_gather(x_hbm, table_hbm, out_hbm, idx_v, rows_v, sem):
    wid = lax.axis_index("s") * NC + lax.axis_index("c")
    base = wid * PER_W

    def body(g, carry):
        off = base + g * CHUNK
        pltpu.sync_copy(x_hbm.at[pl.ds(off, CHUNK)], idx_v)
        pltpu.async_copy(table_hbm.at[idx_v], rows_v, sem).wait()
        pltpu.sync_copy(rows_v, out_hbm.at[pl.ds(off, CHUNK)])
        return carry

    lax.fori_loop(0, STEPS, body, 0)


def kernel(x, table):
    flat = x.reshape(N)
    out = _sc_gather(flat, table)
    return out.reshape(BATCH, SEQ, D)


# speedup vs baseline: 1.1949x; 1.1949x over previous
import functools

import jax
import jax.numpy as jnp
from jax import lax
from jax.experimental import pallas as pl
from jax.experimental.pallas import tpu as pltpu
from jax.experimental.pallas import tpu_sc as plsc

BATCH, SEQ, D = 4096, 200, 64
N = BATCH * SEQ
NC, NS = 2, 16
NW = NC * NS
PER_W = N // NW
CHUNK = 128
CROWS = PER_W // CHUNK
K = 4
SUPER = K * CHUNK
OUTER = PER_W // SUPER
T_HALF = OUTER // 2

_MESH = plsc.VectorSubcoreMesh(core_axis_name="c", subcore_axis_name="s")


@functools.partial(
    pl.kernel,
    out_type=jax.ShapeDtypeStruct((N, D), jnp.float32),
    mesh=_MESH,
    scratch_types=[
        pltpu.VMEM((CROWS, CHUNK), jnp.int32),
        pltpu.VMEM((2, SUPER, D), jnp.float32),
        pltpu.SemaphoreType.DMA,
        pltpu.SemaphoreType.DMA,
        pltpu.SemaphoreType.DMA,
        pltpu.SemaphoreType.DMA,
    ],
    compiler_params=pltpu.CompilerParams(use_tc_tiling_on_sc=False),
)
def _sc_gather(x2_hbm, table_hbm, out_hbm, idx_v, rows_v, sg0, sg1, so0, so1):
    wid = lax.axis_index("s") * NC + lax.axis_index("c")
    base = wid * PER_W
    sem_g = (sg0, sg1)
    sem_o = (so0, so1)

    pltpu.sync_copy(x2_hbm.at[pl.ds(wid * CROWS, CROWS)], idx_v)

    def fire(h, p):
        for j in range(K):
            pltpu.make_async_copy(
                table_hbm.at[idx_v.at[h * K + j]],
                rows_v.at[p, pl.ds(j * CHUNK, CHUNK)],
                sem_g[p],
            ).start()

    def drain(p):
        for j in range(K):
            pltpu.make_async_copy(
                table_hbm.at[idx_v.at[0]],
                rows_v.at[p, pl.ds(j * CHUNK, CHUNK)],
                sem_g[p],
            ).wait()

    def write(h, p):
        pltpu.make_async_copy(
            rows_v.at[p],
            out_hbm.at[pl.ds(base + h * SUPER, SUPER)],
            sem_o[p],
        ).start()

    def wait_write(h, p):
        pltpu.make_async_copy(
            rows_v.at[p],
            out_hbm.at[pl.ds(base + h * SUPER, SUPER)],
            sem_o[p],
        ).wait()

    def body(t, carry):
        h0 = 2 * t
        h1 = h0 + 1

        @pl.when(t >= 1)
        def _():
            wait_write(h0 - 2, 0)
        fire(h0, 0)

        @pl.when(t >= 1)
        def _():
            drain(1)
            write(h0 - 1, 1)

        @pl.when(t >= 1)
        def _():
            wait_write(h1 - 2, 1)
        fire(h1, 1)
        drain(0)
        write(h0, 0)
        return carry

    lax.fori_loop(0, T_HALF, body, 0)

    drain(1)
    write(OUTER - 1, 1)
    wait_write(OUTER - 2, 0)
    wait_write(OUTER - 1, 1)


def kernel(x, table):
    x2 = x.reshape(N // CHUNK, CHUNK)
    out = _sc_gather(x2, table)
    return out.reshape(BATCH, SEQ, D)
